# baseline (speedup 1.0000x reference)
import functools

import numpy as np

import jax
import jax.numpy as jnp
from jax.experimental import pallas as pl
from jax.experimental.pallas import tpu as pltpu


@functools.lru_cache(maxsize=None)
def _gumbel_const(shape, dtype_name):
    with jax.ensure_compile_time_eval():
        g = jax.random.gumbel(jax.random.key(42), shape,
                              dtype=jnp.dtype(dtype_name))
        return np.asarray(jax.device_get(g))


def _router_kernel(x_ref, w_ref, b_ref, g_ref, out_ref, acc_ref):
    i = pl.program_id(0)

    @pl.when(i == 0)
    def _init():
        acc_ref[...] = jnp.zeros_like(acc_ref)

    acc_ref[...] += jnp.sum(x_ref[...], axis=1)

    @pl.when(i == pl.num_programs(0) - 1)
    def _finish():
        s_total = x_ref.shape[1] * pl.num_programs(0)
        z = acc_ref[...] * (1.0 / s_total)
        logits = jax.lax.dot_general(
            z, w_ref[...], (((1,), (1,)), ((), ())),
            preferred_element_type=jnp.float32,
        )
        a = (logits + b_ref[...]) + g_ref[...]
        m = jnp.max(a, axis=-1, keepdims=True)
        e = jnp.exp(a - m)
        y = e / jnp.sum(e, axis=-1, keepdims=True)
        ymax = jnp.max(y, axis=-1, keepdims=True)
        iota = jax.lax.broadcasted_iota(jnp.int32, y.shape, 1)
        idx = jnp.min(jnp.where(y >= ymax, iota, y.shape[-1]), axis=-1,
                      keepdims=True)
        y_hard = (iota == idx).astype(y.dtype)
        out_ref[...] = (y_hard - y) + y


def kernel(x, W, b):
    B, S, D = x.shape
    E = W.shape[0]
    g = jnp.asarray(_gumbel_const((B, E), str(x.dtype)))
    b2 = b.reshape(1, E)

    s_blk = 256
    grid = (S // s_blk,)

    return pl.pallas_call(
        _router_kernel,
        grid=grid,
        in_specs=[
            pl.BlockSpec((B, s_blk, D), lambda i: (0, i, 0)),
            pl.BlockSpec((E, D), lambda i: (0, 0)),
            pl.BlockSpec((1, E), lambda i: (0, 0)),
            pl.BlockSpec((B, E), lambda i: (0, 0)),
        ],
        out_specs=pl.BlockSpec((B, E), lambda i: (0, 0)),
        out_shape=jax.ShapeDtypeStruct((B, E), x.dtype),
        scratch_shapes=[pltpu.VMEM((B, D), jnp.float32)],
        compiler_params=pltpu.CompilerParams(
            dimension_semantics=("arbitrary",),
        ),
    )(x, W, b2, g)
